# Initial kernel scaffold; baseline (speedup 1.0000x reference)
import functools

import jax
import jax.numpy as jnp
from jax import lax
from jax.experimental import pallas as pl
from jax.experimental.pallas import tpu as pltpu
from jax.experimental.pallas import tpu_sc as plsc

NP = 50048
NW = 32
RPT = NP // NW
K = 128


def _probe_call(gt, srcs, dwf, offs):
    mesh = plsc.VectorSubcoreMesh(core_axis_name="c", subcore_axis_name="s")

    @functools.partial(
        pl.kernel,
        out_type=jax.ShapeDtypeStruct((NP, 64), jnp.float32),
        mesh=mesh,
        scratch_types=[
            pltpu.VMEM((RPT, 64), jnp.float32),
            pltpu.VMEM((K,), jnp.int32),
            pltpu.VMEM((K, 64), jnp.float32),
            pltpu.SMEM((K, 2), jnp.float32),
            pltpu.SMEM((48,), jnp.int32),
            pltpu.SemaphoreType.DMA,
        ],
    )
    def k(gt_h, srcs_h, dwf_h, offs_h, out_h, acc, idxb, rowb, dwb, offb, sem):
        wid = lax.axis_index("s") * 2 + lax.axis_index("c")
        base = wid * RPT
        pltpu.sync_copy(offs_h, offb)
        lo = offb[wid]
        hi = offb[wid + 1]
        zeros16 = jnp.zeros((16,), jnp.float32)

        def zbody(i, _):
            for c in range(4):
                acc[i, pl.ds(c * 16, 16)] = zeros16
            return 0

        lax.fori_loop(0, RPT, zbody, 0)
        lo_al = (lo // 8) * 8
        nchunks = (hi - lo_al + K - 1) // K

        def chunk(j, _):
            c0 = lo_al + j * K
            pltpu.sync_copy(srcs_h.at[pl.ds(c0, K)], idxb)
            pltpu.sync_copy(dwf_h.at[pl.ds(c0, K)], dwb)
            pltpu.async_copy(gt_h.at[idxb], rowb, sem).wait()
            k0 = jnp.maximum(lo - c0, 0)
            k1 = jnp.minimum(hi - c0, K)

            def ebody(e, _):
                d = dwb[e, 0].astype(jnp.int32) - base
                wgt = dwb[e, 1]
                for c in range(4):
                    vec = rowb[e, pl.ds(c * 16, 16)] * wgt
                    plsc.addupdate(acc.at[d, pl.ds(c * 16, 16)], vec)
                return 0

            lax.fori_loop(k0, k1, ebody, 0)
            return 0

        lax.fori_loop(0, nchunks, chunk, 0)
        pltpu.sync_copy(acc, out_h.at[pl.ds(base, RPT)])

    return k(gt, srcs, dwf, offs)


def kernel(x, edge_index, edge_weight, Ws, bs, gammas, betas):
    E = edge_index.shape[1]
    src = edge_index[0].astype(jnp.int32)
    dst = edge_index[1].astype(jnp.int32)
    order = jnp.argsort(dst)
    src_s = src[order]
    dst_s = dst[order]
    w_s = edge_weight[order]
    pad = 2 * K
    src_p = jnp.concatenate([src_s, jnp.zeros((pad,), jnp.int32)])
    dstloc = (dst_s % RPT).astype(jnp.float32)
    dwf = jnp.stack(
        [
            jnp.concatenate([dstloc, jnp.zeros((pad,), jnp.float32)]),
            jnp.concatenate([w_s, jnp.zeros((pad,), jnp.float32)]),
        ],
        axis=1,
    )
    offs = jnp.searchsorted(dst_s, jnp.arange(33, dtype=jnp.int32) * RPT).astype(jnp.int32)
    offs = jnp.concatenate([offs, jnp.full((15,), E, jnp.int32)])
    gt = jnp.zeros((NP, 64), jnp.float32)
    r = _probe_call(gt, src_p, dwf, offs)
    return r[:50000, :16]


# speedup vs baseline: 1.7633x; 1.7633x over previous
import functools

import jax
import jax.numpy as jnp
from jax import lax
from jax.experimental import pallas as pl
from jax.experimental.pallas import tpu as pltpu
from jax.experimental.pallas import tpu_sc as plsc

N = 50000
NP = 50176
NW = 32
RPT = NP // NW
K = 128
EPS = 1e-5


def _wid():
    return lax.axis_index("s") * 2 + lax.axis_index("c")


def _sc_mesh():
    return plsc.VectorSubcoreMesh(
        core_axis_name="c", subcore_axis_name="s", num_cores=2, num_subcores=16
    )


def _sc_deg(dloci, wgts, offs):
    @functools.partial(
        pl.kernel,
        out_type=jax.ShapeDtypeStruct((NP, 16), jnp.float32),
        mesh=_sc_mesh(),
        compiler_params=pltpu.CompilerParams(use_tc_tiling_on_sc=False),
        scratch_types=[
            pltpu.VMEM((RPT + 1, 16), jnp.float32),
            pltpu.VMEM((K,), jnp.int32),
            pltpu.VMEM((K,), jnp.float32),
            pltpu.VMEM((48,), jnp.int32),
            pltpu.VMEM((16,), jnp.float32),
        ],
    )
    def k(dloci_h, w_h, offs_h, out_h, acc, dib, wb, offb, cbuf):
        wid = _wid()
        base = wid * RPT
        pltpu.sync_copy(offs_h, offb)
        ovec = offb[pl.ds(wid, 16)]
        lo = ovec[0]
        hi = ovec[1]
        zeros16 = jnp.zeros((16,), jnp.float32)

        def zbody(i, _):
            acc[i, pl.ds(0, 16)] = zeros16
            return 0

        lax.fori_loop(0, RPT, zbody, 0)
        lo_al = (lo // 8) * 8
        nchunks = (hi - lo_al + K - 1) // K
        lanes = lax.iota(jnp.int32, 16)

        cbuf[pl.ds(0, 16)] = zeros16

        def chunk(j, dprev):
            csum = cbuf[pl.ds(0, 16)]
            c0 = lo_al + j * K
            pltpu.sync_copy(dloci_h.at[pl.ds(c0, K)], dib)
            pltpu.sync_copy(w_h.at[pl.ds(c0, K)], wb)
            for g in range(K // 16):
                ev = c0 + g * 16 + lanes
                valid = (ev >= lo) & (ev < hi)
                wvec = jnp.where(valid, wb[pl.ds(g * 16, 16)], 0.0)
                dvec = jnp.where(valid, dib[pl.ds(g * 16, 16)], RPT)
                for l in range(16):
                    d = dvec[l]
                    keep = jnp.where(d == dprev, 1.0, 0.0)
                    csum = csum * jnp.full((16,), keep) + jnp.full((16,), wvec[l])
                    acc[d, pl.ds(0, 16)] = csum
                    dprev = d
            cbuf[pl.ds(0, 16)] = csum
            return dprev

        lax.fori_loop(0, nchunks, chunk, jnp.int32(-1))
        pltpu.sync_copy(acc.at[pl.ds(0, RPT)], out_h.at[pl.ds(base, RPT)])

    return k(dloci, wgts, offs)


def _sc_agg(gt, srcs, dloci, wgts, offs, d_feat):
    cg = d_feat // 16

    @functools.partial(
        pl.kernel,
        out_type=jax.ShapeDtypeStruct((NP, d_feat), jnp.float32),
        mesh=_sc_mesh(),
        compiler_params=pltpu.CompilerParams(use_tc_tiling_on_sc=False),
        scratch_types=[
            pltpu.VMEM((RPT + 1, d_feat), jnp.float32),
            pltpu.VMEM((K,), jnp.int32),
            pltpu.VMEM((K, d_feat), jnp.float32),
            pltpu.VMEM((K,), jnp.int32),
            pltpu.VMEM((K,), jnp.float32),
            pltpu.VMEM((48,), jnp.int32),
            pltpu.SemaphoreType.DMA,
            pltpu.VMEM((64,), jnp.float32),
        ],
    )
    def k(gt_h, srcs_h, dloci_h, w_h, offs_h, out_h, acc, idxb, rowb, dib, wb, offb, sem, cbuf):
        wid = _wid()
        base = wid * RPT
        pltpu.sync_copy(offs_h, offb)
        ovec = offb[pl.ds(wid, 16)]
        lo = ovec[0]
        hi = ovec[1]
        zeros16 = jnp.zeros((16,), jnp.float32)

        def zbody(i, _):
            for c in range(cg):
                acc[i, pl.ds(c * 16, 16)] = zeros16
            return 0

        lax.fori_loop(0, RPT, zbody, 0)
        lo_al = (lo // 8) * 8
        nchunks = (hi - lo_al + K - 1) // K
        lanes = lax.iota(jnp.int32, 16)

        for c in range(cg):
            cbuf[pl.ds(c * 16, 16)] = zeros16

        def chunk(j, dprev):
            csum = [cbuf[pl.ds(c * 16, 16)] for c in range(cg)]
            c0 = lo_al + j * K
            pltpu.sync_copy(srcs_h.at[pl.ds(c0, K)], idxb)
            pltpu.sync_copy(dloci_h.at[pl.ds(c0, K)], dib)
            pltpu.sync_copy(w_h.at[pl.ds(c0, K)], wb)
            pltpu.async_copy(gt_h.at[idxb], rowb, sem).wait()
            for g in range(K // 16):
                ev = c0 + g * 16 + lanes
                valid = (ev >= lo) & (ev < hi)
                wvec = jnp.where(valid, wb[pl.ds(g * 16, 16)], 0.0)
                dvec = jnp.where(valid, dib[pl.ds(g * 16, 16)], RPT)
                for l in range(16):
                    d = dvec[l]
                    wgt = wvec[l]
                    e = g * 16 + l
                    keepv = jnp.full((16,), jnp.where(d == dprev, 1.0, 0.0))
                    ncs = []
                    for c in range(cg):
                        ncs.append(csum[c] * keepv + rowb[e, pl.ds(c * 16, 16)] * wgt)
                        acc[d, pl.ds(c * 16, 16)] = ncs[c]
                    csum = ncs
                    dprev = d
            for c in range(cg):
                cbuf[pl.ds(c * 16, 16)] = csum[c]
            return dprev

        lax.fori_loop(0, nchunks, chunk, jnp.int32(-1))
        pltpu.sync_copy(acc.at[pl.ds(0, RPT)], out_h.at[pl.ds(base, RPT)])

    return k(gt, srcs, dloci, wgts, offs)


NB = 8
BR = NP // NB


def _block_mask(i):
    return (i * BR + lax.broadcasted_iota(jnp.int32, (BR, 1), 0)) < N


def _stats_body(r_ref, b_ref, s_ref, s2_ref):
    i = pl.program_id(0)
    v = r_ref[...] + b_ref[...]
    vm = jnp.where(_block_mask(i), v, 0.0)
    ps = jnp.sum(vm, axis=0, keepdims=True)
    ps2 = jnp.sum(vm * vm, axis=0, keepdims=True)

    @pl.when(i == 0)
    def _():
        s_ref[...] = ps
        s2_ref[...] = ps2

    @pl.when(i > 0)
    def _():
        s_ref[...] += ps
        s2_ref[...] += ps2


def _tc_stats(r, b):
    d = r.shape[1]
    return pl.pallas_call(
        _stats_body,
        grid=(NB,),
        in_specs=[
            pl.BlockSpec((BR, d), lambda i: (i, 0)),
            pl.BlockSpec((1, d), lambda i: (0, 0)),
        ],
        out_specs=(
            pl.BlockSpec((1, d), lambda i: (0, 0)),
            pl.BlockSpec((1, d), lambda i: (0, 0)),
        ),
        out_shape=(
            jax.ShapeDtypeStruct((1, d), jnp.float32),
            jax.ShapeDtypeStruct((1, d), jnp.float32),
        ),
    )(r, b)


def _bn_from_stats(v, s, s2, gamma, beta):
    m = s * (1.0 / N)
    var = jnp.maximum(s2 * (1.0 / N) - m * m, 0.0)
    return gamma * (v - m) / jnp.sqrt(var + EPS) + beta


def _mid_body(r_ref, b_ref, s_ref, s2_ref, gam_ref, bet_ref, w_ref, g_ref):
    v = r_ref[...] + b_ref[...]
    u = _bn_from_stats(v, s_ref[...], s2_ref[...], gam_ref[...], bet_ref[...])
    u = jnp.maximum(u, 0.0)
    g_ref[...] = jnp.dot(u, w_ref[...], preferred_element_type=jnp.float32)


def _tc_mid(r, b, gamma, beta, W):
    d = r.shape[1]
    dn = W.shape[1]
    s, s2 = _tc_stats(r, b)
    return pl.pallas_call(
        _mid_body,
        grid=(NB,),
        in_specs=[
            pl.BlockSpec((BR, d), lambda i: (i, 0)),
            pl.BlockSpec((1, d), lambda i: (0, 0)),
            pl.BlockSpec((1, d), lambda i: (0, 0)),
            pl.BlockSpec((1, d), lambda i: (0, 0)),
            pl.BlockSpec((1, d), lambda i: (0, 0)),
            pl.BlockSpec((1, d), lambda i: (0, 0)),
            pl.BlockSpec((d, dn), lambda i: (0, 0)),
        ],
        out_specs=pl.BlockSpec((BR, dn), lambda i: (i, 0)),
        out_shape=jax.ShapeDtypeStruct((NP, dn), jnp.float32),
    )(r, b, s, s2, gamma, beta, W)


def _first_body(x_ref, s_ref, s2_ref, gam_ref, bet_ref, w_ref, g_ref):
    v = _bn_from_stats(x_ref[...], s_ref[...], s2_ref[...], gam_ref[...], bet_ref[...])
    w0 = w_ref[...]
    g_ref[...] = v[:, 0:1] * w0[0:1, :] + v[:, 1:2] * w0[1:2, :]


def _tc_first(x_p, gamma0, beta0, W0):
    zb = jnp.zeros((1, 2), jnp.float32)
    s, s2 = _tc_stats(x_p, zb)
    return pl.pallas_call(
        _first_body,
        grid=(NB,),
        in_specs=[
            pl.BlockSpec((BR, 2), lambda i: (i, 0)),
            pl.BlockSpec((1, 2), lambda i: (0, 0)),
            pl.BlockSpec((1, 2), lambda i: (0, 0)),
            pl.BlockSpec((1, 2), lambda i: (0, 0)),
            pl.BlockSpec((1, 2), lambda i: (0, 0)),
            pl.BlockSpec((2, 64), lambda i: (0, 0)),
        ],
        out_specs=pl.BlockSpec((BR, 64), lambda i: (i, 0)),
        out_shape=jax.ShapeDtypeStruct((NP, 64), jnp.float32),
    )(x_p, s, s2, gamma0, beta0, W0)


def _last_body(r_ref, b_ref, s_ref, s2_ref, gam_ref, bet_ref, out_ref):
    v = r_ref[...] + b_ref[...]
    out_ref[...] = _bn_from_stats(v, s_ref[...], s2_ref[...], gam_ref[...], bet_ref[...])


def _tc_last(r, b, gamma, beta):
    s, s2 = _tc_stats(r, b)
    return pl.pallas_call(
        _last_body,
        grid=(NB,),
        in_specs=[
            pl.BlockSpec((BR, 16), lambda i: (i, 0)),
            pl.BlockSpec((1, 16), lambda i: (0, 0)),
            pl.BlockSpec((1, 16), lambda i: (0, 0)),
            pl.BlockSpec((1, 16), lambda i: (0, 0)),
            pl.BlockSpec((1, 16), lambda i: (0, 0)),
            pl.BlockSpec((1, 16), lambda i: (0, 0)),
        ],
        out_specs=pl.BlockSpec((BR, 16), lambda i: (i, 0)),
        out_shape=jax.ShapeDtypeStruct((NP, 16), jnp.float32),
    )(r, b, s, s2, gamma, beta)


def kernel(x, edge_index, edge_weight, Ws, bs, gammas, betas):
    E = edge_index.shape[1]
    src = edge_index[0].astype(jnp.int32)
    dst = edge_index[1].astype(jnp.int32)

    order = jnp.argsort(dst)
    src_s = src[order]
    dst_s = dst[order]
    w_s = edge_weight[order]
    pad = 2 * K
    src_p = jnp.concatenate([src_s, jnp.zeros((pad,), jnp.int32)])
    dloci_p = jnp.concatenate([dst_s % RPT, jnp.zeros((pad,), jnp.int32)])
    w_p = jnp.concatenate([w_s, jnp.zeros((pad,), jnp.float32)])
    offs = jnp.searchsorted(dst_s, jnp.arange(33, dtype=jnp.int32) * RPT).astype(
        jnp.int32
    )
    offs = jnp.concatenate([offs, jnp.full((15,), E, jnp.int32)])

    x_p = jnp.concatenate([x, jnp.zeros((NP - N, 2), jnp.float32)], axis=0)

    deg16 = _sc_deg(dloci_p, w_p, offs)
    deg = deg16[:N, 0]
    dinv = jnp.where(deg > 0, lax.rsqrt(jnp.maximum(deg, 1e-12)), 0.0)
    norm_s = dinv[src_s] * w_s * dinv[dst_s]
    norm_p = jnp.concatenate([norm_s, jnp.zeros((pad,), jnp.float32)])

    g = _tc_first(x_p, gammas[0].reshape(1, 2), betas[0].reshape(1, 2), Ws[0])
    n_layers = len(Ws)
    out = None
    for i in range(n_layers):
        d_out = Ws[i].shape[1]
        r = _sc_agg(g, src_p, dloci_p, norm_p, offs, d_out)
        if i < n_layers - 1:
            g = _tc_mid(
                r,
                bs[i].reshape(1, d_out),
                gammas[i + 1].reshape(1, d_out),
                betas[i + 1].reshape(1, d_out),
                Ws[i + 1],
            )
        else:
            out = _tc_last(
                r,
                bs[i].reshape(1, d_out),
                gammas[i + 1].reshape(1, d_out),
                betas[i + 1].reshape(1, d_out),
            )
    return out[:N]
